# Initial kernel scaffold; baseline (speedup 1.0000x reference)
import jax
import jax.numpy as jnp
from jax.experimental import pallas as pl


def kernel(user_id, item_id, theta_w, a_w, b_w, c_w):
    raise NotImplementedError("write your pallas kernel here")


# speedup vs baseline: 1.2142x; 1.2142x over previous
import functools

import jax
import jax.numpy as jnp
from jax import lax
from jax.experimental import pallas as pl
from jax.experimental.pallas import tpu as pltpu
from jax.experimental.pallas import tpu_sc as plsc

BATCH = 16384
NC = 2
NS = 16
L = 16
NW = NC * NS
BPW = BATCH // NW
CHUNK = 128
NCH = BPW // CHUNK


def _irt_body(theta_hbm, a_hbm, b_hbm, c_hbm, uid_hbm, iid_hbm, out_hbm,
              uid_v, iid_v, th_v, a_v, b_v, c_v, out_v, sem):
    wid = lax.axis_index("s") * NC + lax.axis_index("c")
    base = wid * BPW

    pltpu.sync_copy(uid_hbm.at[pl.ds(base, BPW)], uid_v)
    pltpu.sync_copy(iid_hbm.at[pl.ds(base, BPW)], iid_v)

    copies = []
    for j in range(NCH):
        sl = pl.ds(j * CHUNK, CHUNK)
        copies.append(pltpu.async_copy(theta_hbm.at[uid_v.at[sl]], th_v.at[sl], sem))
        copies.append(pltpu.async_copy(a_hbm.at[iid_v.at[sl]], a_v.at[sl], sem))
        copies.append(pltpu.async_copy(b_hbm.at[iid_v.at[sl]], b_v.at[sl], sem))
        copies.append(pltpu.async_copy(c_hbm.at[iid_v.at[sl]], c_v.at[sl], sem))
    for c in copies:
        c.wait()

    for i in range(BPW // L):
        sl = pl.ds(i * L, L)
        th = th_v[sl]
        a = a_v[sl]
        b = b_v[sl]
        c = c_v[sl]
        cs = 1.0 / (1.0 + jnp.exp(-c))
        s = 1.0 / (1.0 + jnp.exp(-1.702 * a * (th - b)))
        out_v[sl] = cs + (1.0 - cs) * s

    pltpu.sync_copy(out_v, out_hbm.at[pl.ds(base, BPW)])


@jax.jit
def _irt_sc(theta, a_tab, b_tab, c_tab, uid, iid):
    mesh = plsc.VectorSubcoreMesh(core_axis_name="c", subcore_axis_name="s")
    return pl.kernel(
        _irt_body,
        mesh=mesh,
        out_type=jax.ShapeDtypeStruct((BATCH,), jnp.float32),
        scratch_types=[
            pltpu.VMEM((BPW,), jnp.int32),
            pltpu.VMEM((BPW,), jnp.int32),
            pltpu.VMEM((BPW,), jnp.float32),
            pltpu.VMEM((BPW,), jnp.float32),
            pltpu.VMEM((BPW,), jnp.float32),
            pltpu.VMEM((BPW,), jnp.float32),
            pltpu.VMEM((BPW,), jnp.float32),
            pltpu.SemaphoreType.DMA,
        ],
    )(theta, a_tab, b_tab, c_tab, uid, iid)


def kernel(user_id, item_id, theta_w, a_w, b_w, c_w):
    uid = user_id.astype(jnp.int32)
    iid = item_id.astype(jnp.int32)
    return _irt_sc(theta_w.reshape(-1), a_w.reshape(-1), b_w.reshape(-1),
                   c_w.reshape(-1), uid, iid)


# speedup vs baseline: 1.2243x; 1.0083x over previous
import functools

import jax
import jax.numpy as jnp
from jax import lax
from jax.experimental import pallas as pl
from jax.experimental.pallas import tpu as pltpu
from jax.experimental.pallas import tpu_sc as plsc

BATCH = 16384
NC = 2
NS = 16
L = 16
NW = NC * NS
BPW = BATCH // NW
CHUNK = 128
NCH = BPW // CHUNK


def _irt_body(theta_hbm, a_hbm, b_hbm, c_hbm, uid_hbm, iid_hbm, out_hbm,
              uid_v, iid_v, th_v, a_v, b_v, c_v, out_v,
              idx_sem, out_sem, *chunk_sems):
    wid = lax.axis_index("s") * NC + lax.axis_index("c")
    base = wid * BPW

    ic0 = pltpu.async_copy(uid_hbm.at[pl.ds(base, BPW)], uid_v, idx_sem)
    ic1 = pltpu.async_copy(iid_hbm.at[pl.ds(base, BPW)], iid_v, idx_sem)
    ic0.wait()
    ic1.wait()

    copies = []
    for j in range(NCH):
        sl = pl.ds(j * CHUNK, CHUNK)
        sem = chunk_sems[j]
        copies.append((
            pltpu.async_copy(theta_hbm.at[uid_v.at[sl]], th_v.at[sl], sem),
            pltpu.async_copy(a_hbm.at[iid_v.at[sl]], a_v.at[sl], sem),
            pltpu.async_copy(b_hbm.at[iid_v.at[sl]], b_v.at[sl], sem),
            pltpu.async_copy(c_hbm.at[iid_v.at[sl]], c_v.at[sl], sem),
        ))

    outs = []
    for j in range(NCH):
        for c in copies[j]:
            c.wait()
        for i in range(j * (CHUNK // L), (j + 1) * (CHUNK // L)):
            sl = pl.ds(i * L, L)
            th = th_v[sl]
            a = a_v[sl]
            b = b_v[sl]
            c = c_v[sl]
            cs = 1.0 / (1.0 + jnp.exp(-c))
            s = 1.0 / (1.0 + jnp.exp(-1.702 * a * (th - b)))
            out_v[sl] = cs + (1.0 - cs) * s
        osl = pl.ds(j * CHUNK, CHUNK)
        outs.append(pltpu.async_copy(
            out_v.at[osl], out_hbm.at[pl.ds(base + j * CHUNK, CHUNK)], out_sem))
    for o in outs:
        o.wait()


@jax.jit
def _irt_sc(theta, a_tab, b_tab, c_tab, uid, iid):
    mesh = plsc.VectorSubcoreMesh(core_axis_name="c", subcore_axis_name="s")
    return pl.kernel(
        _irt_body,
        mesh=mesh,
        out_type=jax.ShapeDtypeStruct((BATCH,), jnp.float32),
        scratch_types=[
            pltpu.VMEM((BPW,), jnp.int32),
            pltpu.VMEM((BPW,), jnp.int32),
            pltpu.VMEM((BPW,), jnp.float32),
            pltpu.VMEM((BPW,), jnp.float32),
            pltpu.VMEM((BPW,), jnp.float32),
            pltpu.VMEM((BPW,), jnp.float32),
            pltpu.VMEM((BPW,), jnp.float32),
            pltpu.SemaphoreType.DMA,
            pltpu.SemaphoreType.DMA,
        ] + [pltpu.SemaphoreType.DMA] * NCH,
    )(theta, a_tab, b_tab, c_tab, uid, iid)


def kernel(user_id, item_id, theta_w, a_w, b_w, c_w):
    uid = user_id.astype(jnp.int32)
    iid = item_id.astype(jnp.int32)
    return _irt_sc(theta_w.reshape(-1), a_w.reshape(-1), b_w.reshape(-1),
                   c_w.reshape(-1), uid, iid)
